# Initial kernel scaffold; baseline (speedup 1.0000x reference)
import jax
import jax.numpy as jnp
from jax.experimental import pallas as pl


def kernel(x, edge_index, params):
    raise NotImplementedError("write your pallas kernel here")


# speedup vs baseline: 8.5017x; 8.5017x over previous
import functools
import jax
import jax.numpy as jnp
from jax import lax
from jax.experimental import pallas as pl
from jax.experimental.pallas import tpu as pltpu
from jax.experimental.pallas import tpu_sc as plsc

N = 10000
E = 160000
E2 = E + N
D_IN = 128
HID = 128
HEADS = 8
NP = 10240
NB = 512
NBLK = NP // NB
NC = 2
NS = 16
CH = 128
E2P = 172032
PER_TILE_32 = E2P // (NC * NS)
PER_TILE_16 = E2P // NS

_HIGH = jax.lax.Precision.HIGHEST


def _lrelu(v):
    return jnp.where(v > 0, v, 0.2 * v)


def _kin_body(x_ref, w_ref, b_ref, o_ref):
    o_ref[...] = jax.nn.relu(
        jnp.dot(x_ref[...], w_ref[...], precision=_HIGH) + b_ref[...])


def _k_in(xp, w, b):
    return pl.pallas_call(
        _kin_body,
        grid=(NBLK,),
        in_specs=[
            pl.BlockSpec((NB, D_IN), lambda i: (i, 0)),
            pl.BlockSpec((D_IN, HID), lambda i: (0, 0)),
            pl.BlockSpec((1, HID), lambda i: (0, 0)),
        ],
        out_specs=pl.BlockSpec((NB, HID), lambda i: (i, 0)),
        out_shape=jax.ShapeDtypeStruct((NP, HID), jnp.float32),
    )(xp, w, b)


def _kpre_body(h_ref, gw_ref, as_ref, ad_ref, tw_ref, tb_ref,
               hh_ref, at_s_ref, at_d_ref, tp_ref, mx_ref):
    i = pl.program_id(0)
    hh = jnp.dot(h_ref[...], gw_ref[...], precision=_HIGH)
    hh_ref[...] = hh
    a_s = jnp.dot(hh, as_ref[...], precision=_HIGH)
    a_d = jnp.dot(hh, ad_ref[...], precision=_HIGH)
    at_s_ref[...] = a_s
    at_d_ref[...] = a_d
    tp_ref[...] = jnp.dot(h_ref[...], tw_ref[...], precision=_HIGH) + tb_ref[...]
    bm = jnp.max(a_s, axis=0, keepdims=True)

    @pl.when(i == 0)
    def _():
        mx_ref[...] = bm

    @pl.when(i > 0)
    def _():
        mx_ref[...] = jnp.maximum(mx_ref[...], bm)


def _k_pre(h, gat_w, as16, ad16, tp_w, tp_b):
    return pl.pallas_call(
        _kpre_body,
        grid=(NBLK,),
        in_specs=[
            pl.BlockSpec((NB, HID), lambda i: (i, 0)),
            pl.BlockSpec((HID, HEADS * HID), lambda i: (0, 0)),
            pl.BlockSpec((HEADS * HID, 16), lambda i: (0, 0)),
            pl.BlockSpec((HEADS * HID, 16), lambda i: (0, 0)),
            pl.BlockSpec((HID, HID), lambda i: (0, 0)),
            pl.BlockSpec((1, HID), lambda i: (0, 0)),
        ],
        out_specs=[
            pl.BlockSpec((NB, HEADS * HID), lambda i: (i, 0)),
            pl.BlockSpec((NB, 16), lambda i: (i, 0)),
            pl.BlockSpec((NB, 16), lambda i: (i, 0)),
            pl.BlockSpec((NB, HID), lambda i: (i, 0)),
            pl.BlockSpec((1, 16), lambda i: (0, 0)),
        ],
        out_shape=[
            jax.ShapeDtypeStruct((NP, HEADS * HID), jnp.float32),
            jax.ShapeDtypeStruct((NP, 16), jnp.float32),
            jax.ShapeDtypeStruct((NP, 16), jnp.float32),
            jax.ShapeDtypeStruct((NP, HID), jnp.float32),
            jax.ShapeDtypeStruct((1, 16), jnp.float32),
        ],
    )(h, gat_w, as16, ad16, tp_w, tp_b)


def _kcomb_body(s_ref, r_ref):
    r_ref[...] = 1.0 / (s_ref[0] + s_ref[1] + 1e-16)


def _k_combine(s2):
    return pl.pallas_call(
        _kcomb_body,
        out_shape=jax.ShapeDtypeStruct((NP, 16), jnp.float32),
    )(s2)


def _kpost_body(oh_ref, r_ref, tp_ref, gb_ref, lg_ref, lb_ref, o_ref):
    acc = jnp.zeros((NB, HID), jnp.float32)
    for hd in range(HEADS):
        acc = acc + oh_ref[hd] * r_ref[:, hd][:, None]
    g = acc * (1.0 / HEADS) + gb_ref[...]
    z = g + tp_ref[...]
    mu = jnp.mean(z, axis=-1, keepdims=True)
    zc = z - mu
    var = jnp.mean(zc * zc, axis=-1, keepdims=True)
    o_ref[...] = jax.nn.relu(zc / jnp.sqrt(var + 1e-5) * lg_ref[...] + lb_ref[...])


def _k_post(out_heads, r, tp, gb, lg, lb):
    return pl.pallas_call(
        _kpost_body,
        grid=(NBLK,),
        in_specs=[
            pl.BlockSpec((HEADS, NB, HID), lambda i: (0, i, 0)),
            pl.BlockSpec((NB, 16), lambda i: (i, 0)),
            pl.BlockSpec((NB, HID), lambda i: (i, 0)),
            pl.BlockSpec((1, HID), lambda i: (0, 0)),
            pl.BlockSpec((1, HID), lambda i: (0, 0)),
            pl.BlockSpec((1, HID), lambda i: (0, 0)),
        ],
        out_specs=pl.BlockSpec((NB, HID), lambda i: (i, 0)),
        out_shape=jax.ShapeDtypeStruct((NP, HID), jnp.float32),
    )(out_heads, r, tp, gb, lg, lb)


def _kcls_body(h_ref, w1_ref, b1_ref, w2_ref, o_ref, acc_ref):
    i = pl.program_id(0)

    @pl.when(i == 0)
    def _():
        acc_ref[...] = jnp.zeros_like(acc_ref)

    rows = i * NB + lax.broadcasted_iota(jnp.int32, (NB, 1), 0)
    hm = jnp.where(rows < N, h_ref[...], 0.0)
    acc_ref[...] = acc_ref[...] + jnp.sum(hm, axis=0, keepdims=True)

    hg = acc_ref[...] * (1.0 / N)
    z = jax.nn.relu(jnp.dot(hg, w1_ref[...], precision=_HIGH) + b1_ref[...])
    o_ref[...] = jnp.dot(z, w2_ref[...], precision=_HIGH)


def _k_cls(h, w1p, b1p, w2p):
    return pl.pallas_call(
        _kcls_body,
        grid=(NBLK,),
        in_specs=[
            pl.BlockSpec((NB, HID), lambda i: (i, 0)),
            pl.BlockSpec((HID, HID), lambda i: (0, 0)),
            pl.BlockSpec((1, HID), lambda i: (0, 0)),
            pl.BlockSpec((HID, HID), lambda i: (0, 0)),
        ],
        out_specs=pl.BlockSpec((1, HID), lambda i: (0, 0)),
        out_shape=jax.ShapeDtypeStruct((1, HID), jnp.float32),
        scratch_shapes=[pltpu.VMEM((1, HID), jnp.float32)],
    )(h, w1p, b1p, w2p)


_MESH = plsc.VectorSubcoreMesh(
    core_axis_name="c", subcore_axis_name="s", num_cores=NC, num_subcores=NS)
_SC_PARAMS = pltpu.CompilerParams(
    use_tc_tiling_on_sc=False, needs_layout_passes=False)


def _sc_stats_body(as_hbm, ad_hbm, mx_hbm, src_hbm, dst_hbm,
                   ex_hbm, s2_hbm,
                   src_v, dst_v, asr_v, adr_v, exr_v, zb_v, mx_v,
                   s_acc, sem_a, sem_b):
    cid = lax.axis_index("c")
    sid = lax.axis_index("s")
    wid = sid * NC + cid
    tile_rows = NP // NS

    for j in range(CH):
        zb_v[j, :] = jnp.zeros((16,), jnp.float32)
    for j in range(tile_rows // CH):
        pltpu.sync_copy(zb_v, s_acc.at[pl.ds(sid * tile_rows + j * CH, CH)])
    pltpu.sync_copy(mx_hbm, mx_v)
    plsc.subcore_barrier()

    base = wid * PER_TILE_32
    nchunk = PER_TILE_32 // CH

    def chunk(ci, _):
        off = base + ci * CH
        pltpu.sync_copy(src_hbm.at[pl.ds(off, CH)], src_v)
        pltpu.sync_copy(dst_hbm.at[pl.ds(off, CH)], dst_v)
        ca = pltpu.async_copy(as_hbm.at[src_v], asr_v, sem_a)
        cb = pltpu.async_copy(ad_hbm.at[dst_v], adr_v, sem_b)
        ca.wait()
        cb.wait()

        def edge(e, _):
            a = asr_v[e, :]
            b = adr_v[e, :]
            ex = jnp.exp(_lrelu(a + b) - _lrelu(mx_v[:] + b))
            exr_v[e, :] = ex
            return 0

        lax.fori_loop(0, CH, edge, 0)
        pltpu.sync_copy(exr_v, ex_hbm.at[pl.ds(off, CH), :])
        pltpu.sync_copy(exr_v, s_acc.at[dst_v], add=True)
        return 0

    lax.fori_loop(0, nchunk, chunk, 0)
    plsc.subcore_barrier()

    pltpu.sync_copy(
        s_acc.at[pl.ds(sid * tile_rows, tile_rows)],
        s2_hbm.at[cid, pl.ds(sid * tile_rows, tile_rows), :])


def _sc_stats(as16, ad16, mx16, srcp, dstp):
    f = pl.kernel(
        _sc_stats_body,
        out_type=[
            jax.ShapeDtypeStruct((E2P, 16), jnp.float32),
            jax.ShapeDtypeStruct((NC, NP, 16), jnp.float32),
        ],
        mesh=_MESH,
        compiler_params=_SC_PARAMS,
        scratch_types=[
            pltpu.VMEM((CH,), jnp.int32),
            pltpu.VMEM((CH,), jnp.int32),
            pltpu.VMEM((CH, 16), jnp.float32),
            pltpu.VMEM((CH, 16), jnp.float32),
            pltpu.VMEM((CH, 16), jnp.float32),
            pltpu.VMEM((CH, 16), jnp.float32),
            pltpu.VMEM((16,), jnp.float32),
            pltpu.VMEM_SHARED((NP, 16), jnp.float32),
            pltpu.SemaphoreType.DMA,
            pltpu.SemaphoreType.DMA,
        ],
    )
    return f(as16, ad16, mx16, srcp, dstp)


def _sc_agg_body(hh8_hbm, ex_hbm, r_hbm, src_hbm, dst_hbm,
                 al_hbm, oh_hbm,
                 src_v, dst_v, gidx_v, rows_v, exr_v, rr_v, zb_v,
                 acc, sem_a):
    cid = lax.axis_index("c")
    sid = lax.axis_index("s")
    wid = sid * NC + cid
    tile_rows = NP // NS

    base = wid * PER_TILE_32

    def achunk(ci, _):
        off = base + ci * CH
        pltpu.sync_copy(dst_hbm.at[pl.ds(off, CH)], dst_v)
        pltpu.sync_copy(ex_hbm.at[pl.ds(off, CH), :], exr_v)
        pltpu.async_copy(r_hbm.at[dst_v], rr_v, sem_a).wait()

        def edge(e, _):
            exr_v[e, :] = exr_v[e, :] * rr_v[e, :]
            return 0

        lax.fori_loop(0, CH, edge, 0)
        pltpu.sync_copy(exr_v, al_hbm.at[pl.ds(off, CH), :])
        return 0

    lax.fori_loop(0, PER_TILE_32 // CH, achunk, 0)

    for j in range(64):
        for q in range(8):
            zb_v[j, pl.ds(q * 16, 16)] = jnp.zeros((16,), jnp.float32)

    for hl in range(HEADS // NC):
        h_abs = cid * (HEADS // NC) + hl

        def zrow(j, _):
            pltpu.sync_copy(zb_v, acc.at[pl.ds(sid * tile_rows + j * 64, 64)])
            return 0

        lax.fori_loop(0, tile_rows // 64, zrow, 0)
        plsc.subcore_barrier()

        ebase = sid * PER_TILE_16

        def chunk(ci, _):
            off = ebase + ci * CH
            pltpu.sync_copy(src_hbm.at[pl.ds(off, CH)], src_v)
            pltpu.sync_copy(dst_hbm.at[pl.ds(off, CH)], dst_v)
            pltpu.sync_copy(ex_hbm.at[pl.ds(off, CH), :], exr_v)
            for q in range(CH // 16):
                gidx_v[pl.ds(q * 16, 16)] = (
                    src_v[pl.ds(q * 16, 16)] * HEADS + h_abs)
            pltpu.async_copy(hh8_hbm.at[gidx_v], rows_v, sem_a).wait()

            def edge(e, _):
                w = plsc.load_gather(
                    exr_v,
                    [jnp.full((16,), e, jnp.int32),
                     jnp.full((16,), h_abs, jnp.int32)])
                for q in range(HID // 16):
                    rows_v[e, pl.ds(q * 16, 16)] = (
                        rows_v[e, pl.ds(q * 16, 16)] * w)
                return 0

            lax.fori_loop(0, CH, edge, 0)
            pltpu.sync_copy(rows_v, acc.at[dst_v], add=True)
            return 0

        lax.fori_loop(0, PER_TILE_16 // CH, chunk, 0)
        plsc.subcore_barrier()

        pltpu.sync_copy(
            acc.at[pl.ds(sid * tile_rows, tile_rows)],
            oh_hbm.at[h_abs, pl.ds(sid * tile_rows, tile_rows), :])
        plsc.subcore_barrier()


def _sc_agg(hh8, ex16, r16, srcp, dstp):
    f = pl.kernel(
        _sc_agg_body,
        out_type=[
            jax.ShapeDtypeStruct((E2P, 16), jnp.float32),
            jax.ShapeDtypeStruct((HEADS, NP, HID), jnp.float32),
        ],
        mesh=_MESH,
        compiler_params=_SC_PARAMS,
        scratch_types=[
            pltpu.VMEM((CH,), jnp.int32),
            pltpu.VMEM((CH,), jnp.int32),
            pltpu.VMEM((CH,), jnp.int32),
            pltpu.VMEM((CH, HID), jnp.float32),
            pltpu.VMEM((CH, 16), jnp.float32),
            pltpu.VMEM((CH, 16), jnp.float32),
            pltpu.VMEM((64, HID), jnp.float32),
            pltpu.VMEM_SHARED((NP, HID), jnp.float32),
            pltpu.SemaphoreType.DMA,
        ],
    )
    return f(hh8, ex16, r16, srcp, dstp)


def kernel(x, edge_index, params):
    loop = jnp.arange(N, dtype=edge_index.dtype)
    src = jnp.concatenate([edge_index[0], loop])
    dst = jnp.concatenate([edge_index[1], loop])
    pad = jnp.full((E2P - E2,), N, jnp.int32)
    srcp = jnp.concatenate([src.astype(jnp.int32), pad])
    dstp = jnp.concatenate([dst.astype(jnp.int32), pad])

    xp = jnp.pad(x, ((0, NP - N), (0, 0)))
    eye16 = jnp.eye(HEADS, 16, dtype=jnp.float32)

    h = _k_in(xp, params["in_W"], params["in_b"].reshape(1, HID))

    alphas = []
    for lp in params["layers"]:
        as16 = jnp.einsum("hd,hk->hdk", lp["att_src"], eye16).reshape(
            HEADS * HID, 16)
        ad16 = jnp.einsum("hd,hk->hdk", lp["att_dst"], eye16).reshape(
            HEADS * HID, 16)
        hh, at_s, at_d, tp, mx = _k_pre(
            h, lp["gat_W"], as16, ad16, lp["tp_W"], lp["tp_b"].reshape(1, HID))
        ex16, s2 = _sc_stats(at_s, at_d, mx.reshape(16), srcp, dstp)
        r16 = _k_combine(s2)
        al16, out_heads = _sc_agg(
            hh.reshape(NP * HEADS, HID), ex16, r16, srcp, dstp)
        h = _k_post(out_heads, r16, tp,
                    lp["gat_b"].reshape(1, HID),
                    lp["ln_g"].reshape(1, HID),
                    lp["ln_b"].reshape(1, HID))
        alphas.append(al16[:E2, :HEADS])

    w1p = jnp.pad(params["c1_W"], ((0, 0), (0, HID - params["c1_W"].shape[1])))
    b1p = jnp.pad(params["c1_b"], (0, HID - params["c1_b"].shape[0]))
    w2p = jnp.pad(params["c2_W"],
                  ((0, HID - params["c2_W"].shape[0]),
                   (0, HID - params["c2_W"].shape[1])))
    logits = _k_cls(h, w1p, b1p.reshape(1, HID), w2p)[:, :2]
    return (logits, *alphas)


# speedup vs baseline: 13.6096x; 1.6008x over previous
import functools
import jax
import jax.numpy as jnp
from jax import lax
from jax.experimental import pallas as pl
from jax.experimental.pallas import tpu as pltpu
from jax.experimental.pallas import tpu_sc as plsc

N = 10000
E = 160000
E2 = E + N
D_IN = 128
HID = 128
HEADS = 8
NP = 10240
NB = 512
NBLK = NP // NB
NC = 2
NS = 16
CH = 128
E2P = 172032
PER_TILE_32 = E2P // (NC * NS)
PER_TILE_16 = E2P // NS

_HIGH = jax.lax.Precision.HIGHEST


def _lrelu(v):
    return jnp.where(v > 0, v, 0.2 * v)


def _kin_body(x_ref, w_ref, b_ref, o_ref):
    o_ref[...] = jax.nn.relu(
        jnp.dot(x_ref[...], w_ref[...], precision=_HIGH) + b_ref[...])


def _k_in(xp, w, b):
    return pl.pallas_call(
        _kin_body,
        grid=(NBLK,),
        in_specs=[
            pl.BlockSpec((NB, D_IN), lambda i: (i, 0)),
            pl.BlockSpec((D_IN, HID), lambda i: (0, 0)),
            pl.BlockSpec((1, HID), lambda i: (0, 0)),
        ],
        out_specs=pl.BlockSpec((NB, HID), lambda i: (i, 0)),
        out_shape=jax.ShapeDtypeStruct((NP, HID), jnp.float32),
    )(xp, w, b)


def _kpre_body(h_ref, gw_ref, as_ref, ad_ref, tw_ref, tb_ref,
               hh_ref, at_s_ref, at_d_ref, tp_ref, mx_ref):
    i = pl.program_id(0)
    hh = jnp.dot(h_ref[...], gw_ref[...], precision=_HIGH)
    hh_ref[...] = hh
    a_s = jnp.dot(hh, as_ref[...], precision=_HIGH)
    a_d = jnp.dot(hh, ad_ref[...], precision=_HIGH)
    at_s_ref[...] = a_s
    at_d_ref[...] = a_d
    tp_ref[...] = jnp.dot(h_ref[...], tw_ref[...], precision=_HIGH) + tb_ref[...]
    bm = jnp.max(a_s, axis=0, keepdims=True)

    @pl.when(i == 0)
    def _():
        mx_ref[...] = bm

    @pl.when(i > 0)
    def _():
        mx_ref[...] = jnp.maximum(mx_ref[...], bm)


def _k_pre(h, gat_w, as16, ad16, tp_w, tp_b):
    return pl.pallas_call(
        _kpre_body,
        grid=(NBLK,),
        in_specs=[
            pl.BlockSpec((NB, HID), lambda i: (i, 0)),
            pl.BlockSpec((HID, HEADS * HID), lambda i: (0, 0)),
            pl.BlockSpec((HEADS * HID, 16), lambda i: (0, 0)),
            pl.BlockSpec((HEADS * HID, 16), lambda i: (0, 0)),
            pl.BlockSpec((HID, HID), lambda i: (0, 0)),
            pl.BlockSpec((1, HID), lambda i: (0, 0)),
        ],
        out_specs=[
            pl.BlockSpec((NB, HEADS * HID), lambda i: (i, 0)),
            pl.BlockSpec((NB, 16), lambda i: (i, 0)),
            pl.BlockSpec((NB, 16), lambda i: (i, 0)),
            pl.BlockSpec((NB, HID), lambda i: (i, 0)),
            pl.BlockSpec((1, 16), lambda i: (0, 0)),
        ],
        out_shape=[
            jax.ShapeDtypeStruct((NP, HEADS * HID), jnp.float32),
            jax.ShapeDtypeStruct((NP, 16), jnp.float32),
            jax.ShapeDtypeStruct((NP, 16), jnp.float32),
            jax.ShapeDtypeStruct((NP, HID), jnp.float32),
            jax.ShapeDtypeStruct((1, 16), jnp.float32),
        ],
    )(h, gat_w, as16, ad16, tp_w, tp_b)


def _kcomb_body(s_ref, r_ref):
    r_ref[...] = 1.0 / (s_ref[0] + s_ref[1] + 1e-16)


def _k_combine(s2):
    return pl.pallas_call(
        _kcomb_body,
        out_shape=jax.ShapeDtypeStruct((NP, 16), jnp.float32),
    )(s2)


def _kpost_body(oh_ref, r_ref, tp_ref, gb_ref, lg_ref, lb_ref, o_ref):
    acc = jnp.zeros((NB, HID), jnp.float32)
    for hd in range(HEADS):
        acc = acc + oh_ref[hd] * r_ref[:, hd][:, None]
    g = acc * (1.0 / HEADS) + gb_ref[...]
    z = g + tp_ref[...]
    mu = jnp.mean(z, axis=-1, keepdims=True)
    zc = z - mu
    var = jnp.mean(zc * zc, axis=-1, keepdims=True)
    o_ref[...] = jax.nn.relu(zc / jnp.sqrt(var + 1e-5) * lg_ref[...] + lb_ref[...])


def _k_post(out_heads, r, tp, gb, lg, lb):
    return pl.pallas_call(
        _kpost_body,
        grid=(NBLK,),
        in_specs=[
            pl.BlockSpec((HEADS, NB, HID), lambda i: (0, i, 0)),
            pl.BlockSpec((NB, 16), lambda i: (i, 0)),
            pl.BlockSpec((NB, HID), lambda i: (i, 0)),
            pl.BlockSpec((1, HID), lambda i: (0, 0)),
            pl.BlockSpec((1, HID), lambda i: (0, 0)),
            pl.BlockSpec((1, HID), lambda i: (0, 0)),
        ],
        out_specs=pl.BlockSpec((NB, HID), lambda i: (i, 0)),
        out_shape=jax.ShapeDtypeStruct((NP, HID), jnp.float32),
    )(out_heads, r, tp, gb, lg, lb)


def _kcls_body(h_ref, w1_ref, b1_ref, w2_ref, o_ref, acc_ref):
    i = pl.program_id(0)

    @pl.when(i == 0)
    def _():
        acc_ref[...] = jnp.zeros_like(acc_ref)

    rows = i * NB + lax.broadcasted_iota(jnp.int32, (NB, 1), 0)
    hm = jnp.where(rows < N, h_ref[...], 0.0)
    acc_ref[...] = acc_ref[...] + jnp.sum(hm, axis=0, keepdims=True)

    hg = acc_ref[...] * (1.0 / N)
    z = jax.nn.relu(jnp.dot(hg, w1_ref[...], precision=_HIGH) + b1_ref[...])
    o_ref[...] = jnp.dot(z, w2_ref[...], precision=_HIGH)


def _k_cls(h, w1p, b1p, w2p):
    return pl.pallas_call(
        _kcls_body,
        grid=(NBLK,),
        in_specs=[
            pl.BlockSpec((NB, HID), lambda i: (i, 0)),
            pl.BlockSpec((HID, HID), lambda i: (0, 0)),
            pl.BlockSpec((1, HID), lambda i: (0, 0)),
            pl.BlockSpec((HID, HID), lambda i: (0, 0)),
        ],
        out_specs=pl.BlockSpec((1, HID), lambda i: (0, 0)),
        out_shape=jax.ShapeDtypeStruct((1, HID), jnp.float32),
        scratch_shapes=[pltpu.VMEM((1, HID), jnp.float32)],
    )(h, w1p, b1p, w2p)


_MESH = plsc.VectorSubcoreMesh(
    core_axis_name="c", subcore_axis_name="s", num_cores=NC, num_subcores=NS)
_SC_PARAMS = pltpu.CompilerParams(
    use_tc_tiling_on_sc=False, needs_layout_passes=False)


def _sc_stats_body(as_hbm, ad_hbm, mx_hbm, src_hbm, dst_hbm,
                   ex_hbm, s2_hbm,
                   src_v, dst_v, asr_v, adr_v, exr_v, zb_v, mx_v,
                   s_acc, sem_a, sem_b):
    cid = lax.axis_index("c")
    sid = lax.axis_index("s")
    wid = sid * NC + cid
    tile_rows = NP // NS

    for j in range(CH):
        zb_v[j, :] = jnp.zeros((16,), jnp.float32)
    for j in range(tile_rows // CH):
        pltpu.sync_copy(zb_v, s_acc.at[pl.ds(sid * tile_rows + j * CH, CH)])
    pltpu.sync_copy(mx_hbm, mx_v)
    plsc.subcore_barrier()

    base = wid * PER_TILE_32
    nchunk = PER_TILE_32 // CH

    def chunk(ci, _):
        off = base + ci * CH
        pltpu.sync_copy(src_hbm.at[pl.ds(off, CH)], src_v)
        pltpu.sync_copy(dst_hbm.at[pl.ds(off, CH)], dst_v)
        ca = pltpu.async_copy(as_hbm.at[src_v], asr_v, sem_a)
        cb = pltpu.async_copy(ad_hbm.at[dst_v], adr_v, sem_b)
        ca.wait()
        cb.wait()

        def edge(e, _):
            a = asr_v[e, :]
            b = adr_v[e, :]
            ex = jnp.exp(_lrelu(a + b) - _lrelu(mx_v[:] + b))
            exr_v[e, :] = ex
            return 0

        lax.fori_loop(0, CH, edge, 0, unroll=8)
        pltpu.sync_copy(exr_v, ex_hbm.at[pl.ds(off, CH), :])
        pltpu.sync_copy(exr_v, s_acc.at[dst_v], add=True)
        return 0

    lax.fori_loop(0, nchunk, chunk, 0)
    plsc.subcore_barrier()

    pltpu.sync_copy(
        s_acc.at[pl.ds(sid * tile_rows, tile_rows)],
        s2_hbm.at[cid, pl.ds(sid * tile_rows, tile_rows), :])


def _sc_stats(as16, ad16, mx16, srcp, dstp):
    f = pl.kernel(
        _sc_stats_body,
        out_type=[
            jax.ShapeDtypeStruct((E2P, 16), jnp.float32),
            jax.ShapeDtypeStruct((NC, NP, 16), jnp.float32),
        ],
        mesh=_MESH,
        compiler_params=_SC_PARAMS,
        scratch_types=[
            pltpu.VMEM((CH,), jnp.int32),
            pltpu.VMEM((CH,), jnp.int32),
            pltpu.VMEM((CH, 16), jnp.float32),
            pltpu.VMEM((CH, 16), jnp.float32),
            pltpu.VMEM((CH, 16), jnp.float32),
            pltpu.VMEM((CH, 16), jnp.float32),
            pltpu.VMEM((16,), jnp.float32),
            pltpu.VMEM_SHARED((NP, 16), jnp.float32),
            pltpu.SemaphoreType.DMA,
            pltpu.SemaphoreType.DMA,
        ],
    )
    return f(as16, ad16, mx16, srcp, dstp)


_NCH = PER_TILE_16 // CH
_NCHH = _NCH // 2


def _sc_agg_body(gidx_hbm, dst2_hbm, hh8_hbm, ex_hbm, r_hbm,
                 al_hbm, oh_hbm,
                 dst_v, gidxall_v, dstall_v, rows0_v, rows1_v,
                 ex0_v, ex1_v, zb_v,
                 acc, sem_g0, sem_g1, sem_x0, sem_x1, sem_a):
    cid = lax.axis_index("c")
    sid = lax.axis_index("s")
    wid = sid * NC + cid
    tile_rows = NP // NS
    rows = (rows0_v, rows1_v)
    exs = (ex0_v, ex1_v)
    sgs = (sem_g0, sem_g1)
    sxs = (sem_x0, sem_x1)

    base = wid * PER_TILE_32

    def achunk(ci, _):
        off = base + ci * CH
        pltpu.sync_copy(dst2_hbm.at[wid * (PER_TILE_32 // CH) + ci], dst_v)
        pltpu.sync_copy(ex_hbm.at[pl.ds(off, CH), :], ex0_v)
        pltpu.async_copy(r_hbm.at[dst_v], ex1_v, sem_a).wait()

        def edge(e, _):
            ex0_v[e, :] = ex0_v[e, :] * ex1_v[e, :]
            return 0

        lax.fori_loop(0, CH, edge, 0, unroll=8)
        pltpu.sync_copy(ex0_v, al_hbm.at[pl.ds(off, CH), :])
        return 0

    lax.fori_loop(0, PER_TILE_32 // CH, achunk, 0)

    for q in range(8 * 8):
        zb_v[q // 8, pl.ds((q % 8) * 16, 16)] = jnp.zeros((16,), jnp.float32)

    for hl in range(HEADS // NC):
        h_abs = cid * (HEADS // NC) + hl

        def zrow(j, _):
            pltpu.sync_copy(zb_v, acc.at[pl.ds(sid * tile_rows + j * 8, 8)])
            return 0

        lax.fori_loop(0, tile_rows // 8, zrow, 0)
        plsc.subcore_barrier()

        for half in range(2):
            pltpu.sync_copy(
                gidx_hbm.at[h_abs, pl.ds(sid * _NCH + half * _NCHH, _NCHH), :],
                gidxall_v)
            pltpu.sync_copy(
                dst2_hbm.at[pl.ds(sid * _NCH + half * _NCHH, _NCHH), :],
                dstall_v)

            ebase = sid * PER_TILE_16 + half * _NCHH * CH

            pltpu.async_copy(hh8_hbm.at[gidxall_v.at[0]], rows0_v, sem_g0)
            pltpu.async_copy(ex_hbm.at[pl.ds(ebase, CH), :], ex0_v, sem_x0)

            @pl.loop(0, _NCHH, step=2)
            def _chunks(g):
                for b in range(2):
                    ci = g + b
                    nb = 1 - b

                    @pl.when(ci + 1 < _NCHH)
                    def _():
                        pltpu.async_copy(
                            hh8_hbm.at[gidxall_v.at[ci + 1]], rows[nb],
                            sgs[nb])
                        pltpu.async_copy(
                            ex_hbm.at[pl.ds(ebase + (ci + 1) * CH, CH), :],
                            exs[nb], sxs[nb])

                    pltpu.make_async_copy(
                        hh8_hbm.at[gidxall_v.at[ci]], rows[b], sgs[b]).wait()
                    pltpu.make_async_copy(
                        ex_hbm.at[pl.ds(ebase + ci * CH, CH), :],
                        exs[b], sxs[b]).wait()

                    rv = rows[b]
                    ev = exs[b]

                    def edge(e, _):
                        w = plsc.load_gather(
                            ev,
                            [jnp.full((16,), e, jnp.int32),
                             jnp.full((16,), h_abs, jnp.int32)])
                        for q in range(HID // 16):
                            rv[e, pl.ds(q * 16, 16)] = (
                                rv[e, pl.ds(q * 16, 16)] * w)
                        return 0

                    lax.fori_loop(0, CH, edge, 0, unroll=4)
                    pltpu.sync_copy(rv, acc.at[dstall_v.at[ci]], add=True)

        plsc.subcore_barrier()

        pltpu.sync_copy(
            acc.at[pl.ds(sid * tile_rows, tile_rows)],
            oh_hbm.at[h_abs, pl.ds(sid * tile_rows, tile_rows), :])
        plsc.subcore_barrier()


def _sc_agg(gidx_all, dst2d, hh8, ex16, r16):
    f = pl.kernel(
        _sc_agg_body,
        out_type=[
            jax.ShapeDtypeStruct((E2P, 16), jnp.float32),
            jax.ShapeDtypeStruct((HEADS, NP, HID), jnp.float32),
        ],
        mesh=_MESH,
        compiler_params=_SC_PARAMS,
        scratch_types=[
            pltpu.VMEM((CH,), jnp.int32),
            pltpu.VMEM((_NCHH, CH), jnp.int32),
            pltpu.VMEM((_NCHH, CH), jnp.int32),
            pltpu.VMEM((CH, HID), jnp.float32),
            pltpu.VMEM((CH, HID), jnp.float32),
            pltpu.VMEM((CH, 16), jnp.float32),
            pltpu.VMEM((CH, 16), jnp.float32),
            pltpu.VMEM((8, HID), jnp.float32),
            pltpu.VMEM_SHARED((NP, HID), jnp.float32),
            pltpu.SemaphoreType.DMA,
            pltpu.SemaphoreType.DMA,
            pltpu.SemaphoreType.DMA,
            pltpu.SemaphoreType.DMA,
            pltpu.SemaphoreType.DMA,
        ],
    )
    return f(gidx_all, dst2d, hh8, ex16, r16)


def kernel(x, edge_index, params):
    loop = jnp.arange(N, dtype=edge_index.dtype)
    src = jnp.concatenate([edge_index[0], loop])
    dst = jnp.concatenate([edge_index[1], loop])
    pad = jnp.full((E2P - E2,), N, jnp.int32)
    srcp = jnp.concatenate([src.astype(jnp.int32), pad])
    dstp = jnp.concatenate([dst.astype(jnp.int32), pad])

    xp = jnp.pad(x, ((0, NP - N), (0, 0)))
    eye16 = jnp.eye(HEADS, 16, dtype=jnp.float32)
    gidx_all = (srcp[None, :] * HEADS
                + jnp.arange(HEADS, dtype=jnp.int32)[:, None]).reshape(
                    HEADS, E2P // CH, CH)
    dst2d = dstp.reshape(E2P // CH, CH)

    h = _k_in(xp, params["in_W"], params["in_b"].reshape(1, HID))

    alphas = []
    for lp in params["layers"]:
        as16 = jnp.einsum("hd,hk->hdk", lp["att_src"], eye16).reshape(
            HEADS * HID, 16)
        ad16 = jnp.einsum("hd,hk->hdk", lp["att_dst"], eye16).reshape(
            HEADS * HID, 16)
        hh, at_s, at_d, tp, mx = _k_pre(
            h, lp["gat_W"], as16, ad16, lp["tp_W"], lp["tp_b"].reshape(1, HID))
        ex16, s2 = _sc_stats(at_s, at_d, mx.reshape(16), srcp, dstp)
        r16 = _k_combine(s2)
        al16, out_heads = _sc_agg(
            gidx_all, dst2d, hh.reshape(NP * HEADS, HID), ex16, r16)
        h = _k_post(out_heads, r16, tp,
                    lp["gat_b"].reshape(1, HID),
                    lp["ln_g"].reshape(1, HID),
                    lp["ln_b"].reshape(1, HID))
        alphas.append(al16[:E2, :HEADS])

    w1p = jnp.pad(params["c1_W"], ((0, 0), (0, HID - params["c1_W"].shape[1])))
    b1p = jnp.pad(params["c1_b"], (0, HID - params["c1_b"].shape[0]))
    w2p = jnp.pad(params["c2_W"],
                  ((0, HID - params["c2_W"].shape[0]),
                   (0, HID - params["c2_W"].shape[1])))
    logits = _k_cls(h, w1p, b1p.reshape(1, HID), w2p)[:, :2]
    return (logits, *alphas)


# speedup vs baseline: 13.6215x; 1.0009x over previous
import functools
import jax
import jax.numpy as jnp
from jax import lax
from jax.experimental import pallas as pl
from jax.experimental.pallas import tpu as pltpu
from jax.experimental.pallas import tpu_sc as plsc

N = 10000
E = 160000
E2 = E + N
D_IN = 128
HID = 128
HEADS = 8
NP = 10240
NB = 512
NBLK = NP // NB
NC = 2
NS = 16
CH = 128
E2P = 172032
PER_TILE_32 = E2P // (NC * NS)
PER_TILE_16 = E2P // NS

_HIGH = jax.lax.Precision.HIGHEST


def _lrelu(v):
    return jnp.where(v > 0, v, 0.2 * v)


def _kin_body(x_ref, w_ref, b_ref, o_ref):
    o_ref[...] = jax.nn.relu(
        jnp.dot(x_ref[...], w_ref[...], precision=_HIGH) + b_ref[...])


def _k_in(xp, w, b):
    return pl.pallas_call(
        _kin_body,
        grid=(NBLK,),
        in_specs=[
            pl.BlockSpec((NB, D_IN), lambda i: (i, 0)),
            pl.BlockSpec((D_IN, HID), lambda i: (0, 0)),
            pl.BlockSpec((1, HID), lambda i: (0, 0)),
        ],
        out_specs=pl.BlockSpec((NB, HID), lambda i: (i, 0)),
        out_shape=jax.ShapeDtypeStruct((NP, HID), jnp.float32),
    )(xp, w, b)


def _kpre_body(h_ref, gw_ref, as_ref, ad_ref, tw_ref, tb_ref,
               hh_ref, at_s_ref, at_d_ref, tp_ref, mx_ref):
    i = pl.program_id(0)
    hh = jnp.dot(h_ref[...], gw_ref[...], precision=_HIGH)
    hh_ref[...] = hh
    a_s = jnp.dot(hh, as_ref[...], precision=_HIGH)
    a_d = jnp.dot(hh, ad_ref[...], precision=_HIGH)
    at_s_ref[...] = a_s
    at_d_ref[...] = a_d
    tp_ref[...] = jnp.dot(h_ref[...], tw_ref[...], precision=_HIGH) + tb_ref[...]
    bm = jnp.max(a_s, axis=0, keepdims=True)

    @pl.when(i == 0)
    def _():
        mx_ref[...] = bm

    @pl.when(i > 0)
    def _():
        mx_ref[...] = jnp.maximum(mx_ref[...], bm)


def _k_pre(h, gat_w, as16, ad16, tp_w, tp_b):
    return pl.pallas_call(
        _kpre_body,
        grid=(NBLK,),
        in_specs=[
            pl.BlockSpec((NB, HID), lambda i: (i, 0)),
            pl.BlockSpec((HID, HEADS * HID), lambda i: (0, 0)),
            pl.BlockSpec((HEADS * HID, 16), lambda i: (0, 0)),
            pl.BlockSpec((HEADS * HID, 16), lambda i: (0, 0)),
            pl.BlockSpec((HID, HID), lambda i: (0, 0)),
            pl.BlockSpec((1, HID), lambda i: (0, 0)),
        ],
        out_specs=[
            pl.BlockSpec((NB, HEADS * HID), lambda i: (i, 0)),
            pl.BlockSpec((NB, 16), lambda i: (i, 0)),
            pl.BlockSpec((NB, 16), lambda i: (i, 0)),
            pl.BlockSpec((NB, HID), lambda i: (i, 0)),
            pl.BlockSpec((1, 16), lambda i: (0, 0)),
        ],
        out_shape=[
            jax.ShapeDtypeStruct((NP, HEADS * HID), jnp.float32),
            jax.ShapeDtypeStruct((NP, 16), jnp.float32),
            jax.ShapeDtypeStruct((NP, 16), jnp.float32),
            jax.ShapeDtypeStruct((NP, HID), jnp.float32),
            jax.ShapeDtypeStruct((1, 16), jnp.float32),
        ],
    )(h, gat_w, as16, ad16, tp_w, tp_b)


def _kcomb_body(s_ref, r_ref):
    r_ref[...] = 1.0 / (s_ref[0] + s_ref[1] + 1e-16)


def _k_combine(s2):
    return pl.pallas_call(
        _kcomb_body,
        out_shape=jax.ShapeDtypeStruct((NP, 16), jnp.float32),
    )(s2)


def _kpost_body(oh_ref, r_ref, tp_ref, gb_ref, lg_ref, lb_ref, o_ref):
    acc = jnp.zeros((NB, HID), jnp.float32)
    for hd in range(HEADS):
        acc = acc + oh_ref[hd] * r_ref[:, hd][:, None]
    g = acc * (1.0 / HEADS) + gb_ref[...]
    z = g + tp_ref[...]
    mu = jnp.mean(z, axis=-1, keepdims=True)
    zc = z - mu
    var = jnp.mean(zc * zc, axis=-1, keepdims=True)
    o_ref[...] = jax.nn.relu(zc / jnp.sqrt(var + 1e-5) * lg_ref[...] + lb_ref[...])


def _k_post(out_heads, r, tp, gb, lg, lb):
    return pl.pallas_call(
        _kpost_body,
        grid=(NBLK,),
        in_specs=[
            pl.BlockSpec((HEADS, NB, HID), lambda i: (0, i, 0)),
            pl.BlockSpec((NB, 16), lambda i: (i, 0)),
            pl.BlockSpec((NB, HID), lambda i: (i, 0)),
            pl.BlockSpec((1, HID), lambda i: (0, 0)),
            pl.BlockSpec((1, HID), lambda i: (0, 0)),
            pl.BlockSpec((1, HID), lambda i: (0, 0)),
        ],
        out_specs=pl.BlockSpec((NB, HID), lambda i: (i, 0)),
        out_shape=jax.ShapeDtypeStruct((NP, HID), jnp.float32),
    )(out_heads, r, tp, gb, lg, lb)


def _kcls_body(h_ref, w1_ref, b1_ref, w2_ref, o_ref, acc_ref):
    i = pl.program_id(0)

    @pl.when(i == 0)
    def _():
        acc_ref[...] = jnp.zeros_like(acc_ref)

    rows = i * NB + lax.broadcasted_iota(jnp.int32, (NB, 1), 0)
    hm = jnp.where(rows < N, h_ref[...], 0.0)
    acc_ref[...] = acc_ref[...] + jnp.sum(hm, axis=0, keepdims=True)

    hg = acc_ref[...] * (1.0 / N)
    z = jax.nn.relu(jnp.dot(hg, w1_ref[...], precision=_HIGH) + b1_ref[...])
    o_ref[...] = jnp.dot(z, w2_ref[...], precision=_HIGH)


def _k_cls(h, w1p, b1p, w2p):
    return pl.pallas_call(
        _kcls_body,
        grid=(NBLK,),
        in_specs=[
            pl.BlockSpec((NB, HID), lambda i: (i, 0)),
            pl.BlockSpec((HID, HID), lambda i: (0, 0)),
            pl.BlockSpec((1, HID), lambda i: (0, 0)),
            pl.BlockSpec((HID, HID), lambda i: (0, 0)),
        ],
        out_specs=pl.BlockSpec((1, HID), lambda i: (0, 0)),
        out_shape=jax.ShapeDtypeStruct((1, HID), jnp.float32),
        scratch_shapes=[pltpu.VMEM((1, HID), jnp.float32)],
    )(h, w1p, b1p, w2p)


_MESH = plsc.VectorSubcoreMesh(
    core_axis_name="c", subcore_axis_name="s", num_cores=NC, num_subcores=NS)
_SC_PARAMS = pltpu.CompilerParams(
    use_tc_tiling_on_sc=False, needs_layout_passes=False)


def _sc_stats_body(as_hbm, ad_hbm, mx_hbm, src_hbm, dst_hbm,
                   ex_hbm, s2_hbm,
                   src_v, dst_v, asr_v, adr_v, exr_v, zb_v, mx_v,
                   s_acc, sem_a, sem_b):
    cid = lax.axis_index("c")
    sid = lax.axis_index("s")
    wid = sid * NC + cid
    tile_rows = NP // NS

    for j in range(CH):
        zb_v[j, :] = jnp.zeros((16,), jnp.float32)
    for j in range(tile_rows // CH):
        pltpu.sync_copy(zb_v, s_acc.at[pl.ds(sid * tile_rows + j * CH, CH)])
    pltpu.sync_copy(mx_hbm, mx_v)
    plsc.subcore_barrier()

    base = wid * PER_TILE_32
    nchunk = PER_TILE_32 // CH

    def chunk(ci, _):
        off = base + ci * CH
        pltpu.sync_copy(src_hbm.at[pl.ds(off, CH)], src_v)
        pltpu.sync_copy(dst_hbm.at[pl.ds(off, CH)], dst_v)
        ca = pltpu.async_copy(as_hbm.at[src_v], asr_v, sem_a)
        cb = pltpu.async_copy(ad_hbm.at[dst_v], adr_v, sem_b)
        ca.wait()
        cb.wait()

        def edge(e, _):
            a = asr_v[e, :]
            b = adr_v[e, :]
            ex = jnp.exp(_lrelu(a + b) - _lrelu(mx_v[:] + b))
            exr_v[e, :] = ex
            return 0

        lax.fori_loop(0, CH, edge, 0, unroll=8)
        pltpu.sync_copy(exr_v, ex_hbm.at[pl.ds(off, CH), :])
        pltpu.sync_copy(exr_v, s_acc.at[dst_v], add=True)
        return 0

    lax.fori_loop(0, nchunk, chunk, 0)
    plsc.subcore_barrier()

    pltpu.sync_copy(
        s_acc.at[pl.ds(sid * tile_rows, tile_rows)],
        s2_hbm.at[cid, pl.ds(sid * tile_rows, tile_rows), :])


def _sc_stats(as16, ad16, mx16, srcp, dstp):
    f = pl.kernel(
        _sc_stats_body,
        out_type=[
            jax.ShapeDtypeStruct((E2P, 16), jnp.float32),
            jax.ShapeDtypeStruct((NC, NP, 16), jnp.float32),
        ],
        mesh=_MESH,
        compiler_params=_SC_PARAMS,
        scratch_types=[
            pltpu.VMEM((CH,), jnp.int32),
            pltpu.VMEM((CH,), jnp.int32),
            pltpu.VMEM((CH, 16), jnp.float32),
            pltpu.VMEM((CH, 16), jnp.float32),
            pltpu.VMEM((CH, 16), jnp.float32),
            pltpu.VMEM((CH, 16), jnp.float32),
            pltpu.VMEM((16,), jnp.float32),
            pltpu.VMEM_SHARED((NP, 16), jnp.float32),
            pltpu.SemaphoreType.DMA,
            pltpu.SemaphoreType.DMA,
        ],
    )
    return f(as16, ad16, mx16, srcp, dstp)


_NCH = PER_TILE_16 // CH
_NCHH = _NCH // 2


def _sc_agg_body(gidx_hbm, dst2_hbm, hh8_hbm, ex_hbm, r_hbm,
                 al_hbm, oh_hbm,
                 dst_v, gidxall_v, dstall_v, rows0_v, rows1_v,
                 ex0_v, ex1_v, zb_v,
                 acc, sem_g0, sem_g1, sem_x0, sem_x1, sem_s0, sem_s1, sem_a):
    cid = lax.axis_index("c")
    sid = lax.axis_index("s")
    wid = sid * NC + cid
    tile_rows = NP // NS
    rows = (rows0_v, rows1_v)
    exs = (ex0_v, ex1_v)
    sgs = (sem_g0, sem_g1)
    sxs = (sem_x0, sem_x1)
    sss = (sem_s0, sem_s1)

    base = wid * PER_TILE_32

    def achunk(ci, _):
        off = base + ci * CH
        pltpu.sync_copy(dst2_hbm.at[wid * (PER_TILE_32 // CH) + ci], dst_v)
        pltpu.sync_copy(ex_hbm.at[pl.ds(off, CH), :], ex0_v)
        pltpu.async_copy(r_hbm.at[dst_v], ex1_v, sem_a).wait()

        def edge(e, _):
            ex0_v[e, :] = ex0_v[e, :] * ex1_v[e, :]
            return 0

        lax.fori_loop(0, CH, edge, 0, unroll=8)
        pltpu.sync_copy(ex0_v, al_hbm.at[pl.ds(off, CH), :])
        return 0

    lax.fori_loop(0, PER_TILE_32 // CH, achunk, 0)

    for q in range(8 * 8):
        zb_v[q // 8, pl.ds((q % 8) * 16, 16)] = jnp.zeros((16,), jnp.float32)

    for hl in range(HEADS // NC):
        h_abs = cid * (HEADS // NC) + hl

        def zrow(j, _):
            pltpu.sync_copy(zb_v, acc.at[pl.ds(sid * tile_rows + j * 8, 8)])
            return 0

        lax.fori_loop(0, tile_rows // 8, zrow, 0)
        plsc.subcore_barrier()

        for half in range(2):
            pltpu.sync_copy(
                gidx_hbm.at[h_abs, pl.ds(sid * _NCH + half * _NCHH, _NCHH), :],
                gidxall_v)
            pltpu.sync_copy(
                dst2_hbm.at[pl.ds(sid * _NCH + half * _NCHH, _NCHH), :],
                dstall_v)

            ebase = sid * PER_TILE_16 + half * _NCHH * CH

            pltpu.async_copy(hh8_hbm.at[gidxall_v.at[0]], rows0_v, sem_g0)
            pltpu.async_copy(ex_hbm.at[pl.ds(ebase, CH), :], ex0_v, sem_x0)

            @pl.loop(0, _NCHH, step=2)
            def _chunks(g):
                for b in range(2):
                    ci = g + b
                    nb = 1 - b

                    @pl.when(ci + 1 < _NCHH)
                    def _():
                        @pl.when(ci >= 1)
                        def _():
                            pltpu.make_async_copy(
                                rows[nb], acc.at[dstall_v.at[0]],
                                sss[nb]).wait()
                        pltpu.async_copy(
                            hh8_hbm.at[gidxall_v.at[ci + 1]], rows[nb],
                            sgs[nb])
                        pltpu.async_copy(
                            ex_hbm.at[pl.ds(ebase + (ci + 1) * CH, CH), :],
                            exs[nb], sxs[nb])

                    pltpu.make_async_copy(
                        hh8_hbm.at[gidxall_v.at[ci]], rows[b], sgs[b]).wait()
                    pltpu.make_async_copy(
                        ex_hbm.at[pl.ds(ebase + ci * CH, CH), :],
                        exs[b], sxs[b]).wait()

                    rv = rows[b]
                    ev = exs[b]

                    def edge(e, _):
                        w = plsc.load_gather(
                            ev,
                            [jnp.full((16,), e, jnp.int32),
                             jnp.full((16,), h_abs, jnp.int32)])
                        for q in range(HID // 16):
                            rv[e, pl.ds(q * 16, 16)] = (
                                rv[e, pl.ds(q * 16, 16)] * w)
                        return 0

                    lax.fori_loop(0, CH, edge, 0, unroll=4)
                    pltpu.async_copy(
                        rv, acc.at[dstall_v.at[ci]], sss[b], add=True)

            pltpu.make_async_copy(
                rows0_v, acc.at[dstall_v.at[0]], sem_s0).wait()
            pltpu.make_async_copy(
                rows1_v, acc.at[dstall_v.at[0]], sem_s1).wait()

        plsc.subcore_barrier()

        pltpu.sync_copy(
            acc.at[pl.ds(sid * tile_rows, tile_rows)],
            oh_hbm.at[h_abs, pl.ds(sid * tile_rows, tile_rows), :])
        plsc.subcore_barrier()


def _sc_agg(gidx_all, dst2d, hh8, ex16, r16):
    f = pl.kernel(
        _sc_agg_body,
        out_type=[
            jax.ShapeDtypeStruct((E2P, 16), jnp.float32),
            jax.ShapeDtypeStruct((HEADS, NP, HID), jnp.float32),
        ],
        mesh=_MESH,
        compiler_params=_SC_PARAMS,
        scratch_types=[
            pltpu.VMEM((CH,), jnp.int32),
            pltpu.VMEM((_NCHH, CH), jnp.int32),
            pltpu.VMEM((_NCHH, CH), jnp.int32),
            pltpu.VMEM((CH, HID), jnp.float32),
            pltpu.VMEM((CH, HID), jnp.float32),
            pltpu.VMEM((CH, 16), jnp.float32),
            pltpu.VMEM((CH, 16), jnp.float32),
            pltpu.VMEM((8, HID), jnp.float32),
            pltpu.VMEM_SHARED((NP, HID), jnp.float32),
            pltpu.SemaphoreType.DMA,
            pltpu.SemaphoreType.DMA,
            pltpu.SemaphoreType.DMA,
            pltpu.SemaphoreType.DMA,
            pltpu.SemaphoreType.DMA,
            pltpu.SemaphoreType.DMA,
            pltpu.SemaphoreType.DMA,
        ],
    )
    return f(gidx_all, dst2d, hh8, ex16, r16)


def kernel(x, edge_index, params):
    loop = jnp.arange(N, dtype=edge_index.dtype)
    src = jnp.concatenate([edge_index[0], loop])
    dst = jnp.concatenate([edge_index[1], loop])
    pad = jnp.full((E2P - E2,), N, jnp.int32)
    srcp = jnp.concatenate([src.astype(jnp.int32), pad])
    dstp = jnp.concatenate([dst.astype(jnp.int32), pad])

    xp = jnp.pad(x, ((0, NP - N), (0, 0)))
    eye16 = jnp.eye(HEADS, 16, dtype=jnp.float32)
    gidx_all = (srcp[None, :] * HEADS
                + jnp.arange(HEADS, dtype=jnp.int32)[:, None]).reshape(
                    HEADS, E2P // CH, CH)
    dst2d = dstp.reshape(E2P // CH, CH)

    h = _k_in(xp, params["in_W"], params["in_b"].reshape(1, HID))

    alphas = []
    for lp in params["layers"]:
        as16 = jnp.einsum("hd,hk->hdk", lp["att_src"], eye16).reshape(
            HEADS * HID, 16)
        ad16 = jnp.einsum("hd,hk->hdk", lp["att_dst"], eye16).reshape(
            HEADS * HID, 16)
        hh, at_s, at_d, tp, mx = _k_pre(
            h, lp["gat_W"], as16, ad16, lp["tp_W"], lp["tp_b"].reshape(1, HID))
        ex16, s2 = _sc_stats(at_s, at_d, mx.reshape(16), srcp, dstp)
        r16 = _k_combine(s2)
        al16, out_heads = _sc_agg(
            gidx_all, dst2d, hh.reshape(NP * HEADS, HID), ex16, r16)
        h = _k_post(out_heads, r16, tp,
                    lp["gat_b"].reshape(1, HID),
                    lp["ln_g"].reshape(1, HID),
                    lp["ln_b"].reshape(1, HID))
        alphas.append(al16[:E2, :HEADS])

    w1p = jnp.pad(params["c1_W"], ((0, 0), (0, HID - params["c1_W"].shape[1])))
    b1p = jnp.pad(params["c1_b"], (0, HID - params["c1_b"].shape[0]))
    w2p = jnp.pad(params["c2_W"],
                  ((0, HID - params["c2_W"].shape[0]),
                   (0, HID - params["c2_W"].shape[1])))
    logits = _k_cls(h, w1p, b1p.reshape(1, HID), w2p)[:, :2]
    return (logits, *alphas)


# speedup vs baseline: 13.7644x; 1.0105x over previous
import functools
import jax
import jax.numpy as jnp
from jax import lax
from jax.experimental import pallas as pl
from jax.experimental.pallas import tpu as pltpu
from jax.experimental.pallas import tpu_sc as plsc

N = 10000
E = 160000
E2 = E + N
D_IN = 128
HID = 128
HEADS = 8
NP = 10240
NB = 512
NBLK = NP // NB
NC = 2
NS = 16
CH = 128
E2P = 172032
PER_TILE_32 = E2P // (NC * NS)
PER_TILE_16 = E2P // NS

_HIGH = jax.lax.Precision.HIGHEST


def _lrelu(v):
    return jnp.where(v > 0, v, 0.2 * v)


def _kin_body(x_ref, w_ref, b_ref, o_ref):
    o_ref[...] = jax.nn.relu(
        jnp.dot(x_ref[...], w_ref[...], precision=_HIGH) + b_ref[...])


def _k_in(xp, w, b):
    return pl.pallas_call(
        _kin_body,
        grid=(NBLK,),
        in_specs=[
            pl.BlockSpec((NB, D_IN), lambda i: (i, 0)),
            pl.BlockSpec((D_IN, HID), lambda i: (0, 0)),
            pl.BlockSpec((1, HID), lambda i: (0, 0)),
        ],
        out_specs=pl.BlockSpec((NB, HID), lambda i: (i, 0)),
        out_shape=jax.ShapeDtypeStruct((NP, HID), jnp.float32),
    )(xp, w, b)


def _kpre_body(h_ref, gw_ref, as_ref, ad_ref, tw_ref, tb_ref,
               hh_ref, at_s_ref, at_d_ref, tp_ref, mx_ref):
    i = pl.program_id(0)
    hh = jnp.dot(h_ref[...], gw_ref[...], precision=_HIGH)
    hh_ref[...] = hh
    a_s = jnp.dot(hh, as_ref[...], precision=_HIGH)
    a_d = jnp.dot(hh, ad_ref[...], precision=_HIGH)
    at_s_ref[...] = a_s
    at_d_ref[...] = a_d
    tp_ref[...] = jnp.dot(h_ref[...], tw_ref[...], precision=_HIGH) + tb_ref[...]
    bm = jnp.max(a_s, axis=0, keepdims=True)

    @pl.when(i == 0)
    def _():
        mx_ref[...] = bm

    @pl.when(i > 0)
    def _():
        mx_ref[...] = jnp.maximum(mx_ref[...], bm)


def _k_pre(h, gat_w, as16, ad16, tp_w, tp_b):
    return pl.pallas_call(
        _kpre_body,
        grid=(NBLK,),
        in_specs=[
            pl.BlockSpec((NB, HID), lambda i: (i, 0)),
            pl.BlockSpec((HID, HEADS * HID), lambda i: (0, 0)),
            pl.BlockSpec((HEADS * HID, 16), lambda i: (0, 0)),
            pl.BlockSpec((HEADS * HID, 16), lambda i: (0, 0)),
            pl.BlockSpec((HID, HID), lambda i: (0, 0)),
            pl.BlockSpec((1, HID), lambda i: (0, 0)),
        ],
        out_specs=[
            pl.BlockSpec((NB, HEADS * HID), lambda i: (i, 0)),
            pl.BlockSpec((NB, 16), lambda i: (i, 0)),
            pl.BlockSpec((NB, 16), lambda i: (i, 0)),
            pl.BlockSpec((NB, HID), lambda i: (i, 0)),
            pl.BlockSpec((1, 16), lambda i: (0, 0)),
        ],
        out_shape=[
            jax.ShapeDtypeStruct((NP, HEADS * HID), jnp.float32),
            jax.ShapeDtypeStruct((NP, 16), jnp.float32),
            jax.ShapeDtypeStruct((NP, 16), jnp.float32),
            jax.ShapeDtypeStruct((NP, HID), jnp.float32),
            jax.ShapeDtypeStruct((1, 16), jnp.float32),
        ],
    )(h, gat_w, as16, ad16, tp_w, tp_b)


def _kcomb_body(s_ref, r_ref):
    r_ref[...] = 1.0 / (s_ref[0] + s_ref[1] + 1e-16)


def _k_combine(s2):
    return pl.pallas_call(
        _kcomb_body,
        out_shape=jax.ShapeDtypeStruct((NP, 16), jnp.float32),
    )(s2)


def _kpost_body(oh_ref, r_ref, tp_ref, gb_ref, lg_ref, lb_ref, o_ref):
    acc = jnp.zeros((NB, HID), jnp.float32)
    for hd in range(HEADS):
        acc = acc + oh_ref[hd] * r_ref[:, hd][:, None]
    g = acc * (1.0 / HEADS) + gb_ref[...]
    z = g + tp_ref[...]
    mu = jnp.mean(z, axis=-1, keepdims=True)
    zc = z - mu
    var = jnp.mean(zc * zc, axis=-1, keepdims=True)
    o_ref[...] = jax.nn.relu(zc / jnp.sqrt(var + 1e-5) * lg_ref[...] + lb_ref[...])


def _k_post(out_heads, r, tp, gb, lg, lb):
    return pl.pallas_call(
        _kpost_body,
        grid=(NBLK,),
        in_specs=[
            pl.BlockSpec((HEADS, NB, HID), lambda i: (0, i, 0)),
            pl.BlockSpec((NB, 16), lambda i: (i, 0)),
            pl.BlockSpec((NB, HID), lambda i: (i, 0)),
            pl.BlockSpec((1, HID), lambda i: (0, 0)),
            pl.BlockSpec((1, HID), lambda i: (0, 0)),
            pl.BlockSpec((1, HID), lambda i: (0, 0)),
        ],
        out_specs=pl.BlockSpec((NB, HID), lambda i: (i, 0)),
        out_shape=jax.ShapeDtypeStruct((NP, HID), jnp.float32),
    )(out_heads, r, tp, gb, lg, lb)


def _kcls_body(h_ref, w1_ref, b1_ref, w2_ref, o_ref, acc_ref):
    i = pl.program_id(0)

    @pl.when(i == 0)
    def _():
        acc_ref[...] = jnp.zeros_like(acc_ref)

    rows = i * NB + lax.broadcasted_iota(jnp.int32, (NB, 1), 0)
    hm = jnp.where(rows < N, h_ref[...], 0.0)
    acc_ref[...] = acc_ref[...] + jnp.sum(hm, axis=0, keepdims=True)

    hg = acc_ref[...] * (1.0 / N)
    z = jax.nn.relu(jnp.dot(hg, w1_ref[...], precision=_HIGH) + b1_ref[...])
    o_ref[...] = jnp.dot(z, w2_ref[...], precision=_HIGH)


def _k_cls(h, w1p, b1p, w2p):
    return pl.pallas_call(
        _kcls_body,
        grid=(NBLK,),
        in_specs=[
            pl.BlockSpec((NB, HID), lambda i: (i, 0)),
            pl.BlockSpec((HID, HID), lambda i: (0, 0)),
            pl.BlockSpec((1, HID), lambda i: (0, 0)),
            pl.BlockSpec((HID, HID), lambda i: (0, 0)),
        ],
        out_specs=pl.BlockSpec((1, HID), lambda i: (0, 0)),
        out_shape=jax.ShapeDtypeStruct((1, HID), jnp.float32),
        scratch_shapes=[pltpu.VMEM((1, HID), jnp.float32)],
    )(h, w1p, b1p, w2p)


_MESH = plsc.VectorSubcoreMesh(
    core_axis_name="c", subcore_axis_name="s", num_cores=NC, num_subcores=NS)
_SC_PARAMS = pltpu.CompilerParams(
    use_tc_tiling_on_sc=False, needs_layout_passes=False)


def _sc_stats_body(as_hbm, ad_hbm, mx_hbm, src_hbm, dst_hbm,
                   ex_hbm, s2_hbm,
                   src_v, dst_v, asr_v, adr_v, exr_v, zb_v, mx_v,
                   s_acc, sem_a, sem_b):
    cid = lax.axis_index("c")
    sid = lax.axis_index("s")
    wid = sid * NC + cid
    tile_rows = NP // NS

    for j in range(CH):
        zb_v[j, :] = jnp.zeros((16,), jnp.float32)
    for j in range(tile_rows // CH):
        pltpu.sync_copy(zb_v, s_acc.at[pl.ds(sid * tile_rows + j * CH, CH)])
    pltpu.sync_copy(mx_hbm, mx_v)
    plsc.subcore_barrier()

    base = wid * PER_TILE_32
    nchunk = PER_TILE_32 // CH

    def chunk(ci, _):
        off = base + ci * CH
        pltpu.sync_copy(src_hbm.at[pl.ds(off, CH)], src_v)
        pltpu.sync_copy(dst_hbm.at[pl.ds(off, CH)], dst_v)
        ca = pltpu.async_copy(as_hbm.at[src_v], asr_v, sem_a)
        cb = pltpu.async_copy(ad_hbm.at[dst_v], adr_v, sem_b)
        ca.wait()
        cb.wait()

        def edge(e, _):
            a = asr_v[e, :]
            b = adr_v[e, :]
            ex = jnp.exp(_lrelu(a + b) - _lrelu(mx_v[:] + b))
            exr_v[e, :] = ex
            return 0

        lax.fori_loop(0, CH, edge, 0, unroll=8)
        pltpu.sync_copy(exr_v, ex_hbm.at[pl.ds(off, CH), :])
        pltpu.sync_copy(exr_v, s_acc.at[dst_v], add=True)
        return 0

    lax.fori_loop(0, nchunk, chunk, 0)
    plsc.subcore_barrier()

    pltpu.sync_copy(
        s_acc.at[pl.ds(sid * tile_rows, tile_rows)],
        s2_hbm.at[cid, pl.ds(sid * tile_rows, tile_rows), :])


def _sc_stats(as16, ad16, mx16, srcp, dstp):
    f = pl.kernel(
        _sc_stats_body,
        out_type=[
            jax.ShapeDtypeStruct((E2P, 16), jnp.float32),
            jax.ShapeDtypeStruct((NC, NP, 16), jnp.float32),
        ],
        mesh=_MESH,
        compiler_params=_SC_PARAMS,
        scratch_types=[
            pltpu.VMEM((CH,), jnp.int32),
            pltpu.VMEM((CH,), jnp.int32),
            pltpu.VMEM((CH, 16), jnp.float32),
            pltpu.VMEM((CH, 16), jnp.float32),
            pltpu.VMEM((CH, 16), jnp.float32),
            pltpu.VMEM((CH, 16), jnp.float32),
            pltpu.VMEM((16,), jnp.float32),
            pltpu.VMEM_SHARED((NP, 16), jnp.float32),
            pltpu.SemaphoreType.DMA,
            pltpu.SemaphoreType.DMA,
        ],
    )
    return f(as16, ad16, mx16, srcp, dstp)


_NCH = PER_TILE_16 // CH
_NCHH = _NCH // 2


def _sc_agg_body(gidx_hbm, dst2_hbm, hh8_hbm, ex_hbm, r_hbm,
                 al_hbm, oh_hbm,
                 dst_v, gidxall_v, dstall_v, rows0_v, rows1_v,
                 ex0_v, ex1_v,
                 acc, sem_g0, sem_g1, sem_x0, sem_x1, sem_s0, sem_s1, sem_a):
    cid = lax.axis_index("c")
    sid = lax.axis_index("s")
    wid = sid * NC + cid
    tile_rows = NP // NS
    rows = (rows0_v, rows1_v)
    exs = (ex0_v, ex1_v)
    sgs = (sem_g0, sem_g1)
    sxs = (sem_x0, sem_x1)
    sss = (sem_s0, sem_s1)

    base = wid * PER_TILE_32

    def achunk(ci, _):
        off = base + ci * CH
        pltpu.sync_copy(dst2_hbm.at[wid * (PER_TILE_32 // CH) + ci], dst_v)
        pltpu.sync_copy(ex_hbm.at[pl.ds(off, CH), :], ex0_v)
        pltpu.async_copy(r_hbm.at[dst_v], ex1_v, sem_a).wait()

        def edge(e, _):
            ex0_v[e, :] = ex0_v[e, :] * ex1_v[e, :]
            return 0

        lax.fori_loop(0, CH, edge, 0, unroll=8)
        pltpu.sync_copy(ex0_v, al_hbm.at[pl.ds(off, CH), :])
        return 0

    lax.fori_loop(0, PER_TILE_32 // CH, achunk, 0)

    for hl in range(HEADS // NC):
        h_abs = cid * (HEADS // NC) + hl

        def zfill(j, _):
            for q in range(8):
                rows0_v[j, pl.ds(q * 16, 16)] = jnp.zeros((16,), jnp.float32)
            return 0

        lax.fori_loop(0, CH, zfill, 0, unroll=8)

        def zrow(j, _):
            pltpu.async_copy(
                rows0_v, acc.at[pl.ds(sid * tile_rows + j * CH, CH)], sem_a)
            return 0

        lax.fori_loop(0, tile_rows // CH, zrow, 0)

        def zdrain(j, _):
            pltpu.make_async_copy(
                rows0_v, acc.at[pl.ds(sid * tile_rows, CH)], sem_a).wait()
            return 0

        lax.fori_loop(0, tile_rows // CH, zdrain, 0)
        plsc.subcore_barrier()

        for half in range(2):
            pltpu.sync_copy(
                gidx_hbm.at[h_abs, pl.ds(sid * _NCH + half * _NCHH, _NCHH), :],
                gidxall_v)
            pltpu.sync_copy(
                dst2_hbm.at[pl.ds(sid * _NCH + half * _NCHH, _NCHH), :],
                dstall_v)

            ebase = sid * PER_TILE_16 + half * _NCHH * CH

            pltpu.async_copy(hh8_hbm.at[gidxall_v.at[0]], rows0_v, sem_g0)
            pltpu.async_copy(ex_hbm.at[pl.ds(ebase, CH), :], ex0_v, sem_x0)

            @pl.loop(0, _NCHH, step=2)
            def _chunks(g):
                for b in range(2):
                    ci = g + b
                    nb = 1 - b

                    @pl.when(ci + 1 < _NCHH)
                    def _():
                        @pl.when(ci >= 1)
                        def _():
                            pltpu.make_async_copy(
                                rows[nb], acc.at[dstall_v.at[0]],
                                sss[nb]).wait()
                        pltpu.async_copy(
                            hh8_hbm.at[gidxall_v.at[ci + 1]], rows[nb],
                            sgs[nb])
                        pltpu.async_copy(
                            ex_hbm.at[pl.ds(ebase + (ci + 1) * CH, CH), :],
                            exs[nb], sxs[nb])

                    pltpu.make_async_copy(
                        hh8_hbm.at[gidxall_v.at[ci]], rows[b], sgs[b]).wait()
                    pltpu.make_async_copy(
                        ex_hbm.at[pl.ds(ebase + ci * CH, CH), :],
                        exs[b], sxs[b]).wait()

                    rv = rows[b]
                    ev = exs[b]

                    def edge(e, _):
                        w = plsc.load_gather(
                            ev,
                            [jnp.full((16,), e, jnp.int32),
                             jnp.full((16,), h_abs, jnp.int32)])
                        for q in range(HID // 16):
                            rv[e, pl.ds(q * 16, 16)] = (
                                rv[e, pl.ds(q * 16, 16)] * w)
                        return 0

                    lax.fori_loop(0, CH, edge, 0, unroll=4)
                    pltpu.async_copy(
                        rv, acc.at[dstall_v.at[ci]], sss[b], add=True)

            pltpu.make_async_copy(
                rows0_v, acc.at[dstall_v.at[0]], sem_s0).wait()
            pltpu.make_async_copy(
                rows1_v, acc.at[dstall_v.at[0]], sem_s1).wait()

        plsc.subcore_barrier()

        pltpu.sync_copy(
            acc.at[pl.ds(sid * tile_rows, tile_rows)],
            oh_hbm.at[h_abs, pl.ds(sid * tile_rows, tile_rows), :])
        plsc.subcore_barrier()


def _sc_agg(gidx_all, dst2d, hh8, ex16, r16):
    f = pl.kernel(
        _sc_agg_body,
        out_type=[
            jax.ShapeDtypeStruct((E2P, 16), jnp.float32),
            jax.ShapeDtypeStruct((HEADS, NP, HID), jnp.float32),
        ],
        mesh=_MESH,
        compiler_params=_SC_PARAMS,
        scratch_types=[
            pltpu.VMEM((CH,), jnp.int32),
            pltpu.VMEM((_NCHH, CH), jnp.int32),
            pltpu.VMEM((_NCHH, CH), jnp.int32),
            pltpu.VMEM((CH, HID), jnp.float32),
            pltpu.VMEM((CH, HID), jnp.float32),
            pltpu.VMEM((CH, 16), jnp.float32),
            pltpu.VMEM((CH, 16), jnp.float32),
            pltpu.VMEM_SHARED((NP, HID), jnp.float32),
            pltpu.SemaphoreType.DMA,
            pltpu.SemaphoreType.DMA,
            pltpu.SemaphoreType.DMA,
            pltpu.SemaphoreType.DMA,
            pltpu.SemaphoreType.DMA,
            pltpu.SemaphoreType.DMA,
            pltpu.SemaphoreType.DMA,
        ],
    )
    return f(gidx_all, dst2d, hh8, ex16, r16)


def kernel(x, edge_index, params):
    loop = jnp.arange(N, dtype=edge_index.dtype)
    src = jnp.concatenate([edge_index[0], loop])
    dst = jnp.concatenate([edge_index[1], loop])
    pad = jnp.full((E2P - E2,), N, jnp.int32)
    srcp = jnp.concatenate([src.astype(jnp.int32), pad])
    dstp = jnp.concatenate([dst.astype(jnp.int32), pad])

    xp = jnp.pad(x, ((0, NP - N), (0, 0)))
    eye16 = jnp.eye(HEADS, 16, dtype=jnp.float32)
    gidx_all = (srcp[None, :] * HEADS
                + jnp.arange(HEADS, dtype=jnp.int32)[:, None]).reshape(
                    HEADS, E2P // CH, CH)
    dst2d = dstp.reshape(E2P // CH, CH)

    h = _k_in(xp, params["in_W"], params["in_b"].reshape(1, HID))

    alphas = []
    for lp in params["layers"]:
        as16 = jnp.einsum("hd,hk->hdk", lp["att_src"], eye16).reshape(
            HEADS * HID, 16)
        ad16 = jnp.einsum("hd,hk->hdk", lp["att_dst"], eye16).reshape(
            HEADS * HID, 16)
        hh, at_s, at_d, tp, mx = _k_pre(
            h, lp["gat_W"], as16, ad16, lp["tp_W"], lp["tp_b"].reshape(1, HID))
        ex16, s2 = _sc_stats(at_s, at_d, mx.reshape(16), srcp, dstp)
        r16 = _k_combine(s2)
        al16, out_heads = _sc_agg(
            gidx_all, dst2d, hh.reshape(NP * HEADS, HID), ex16, r16)
        h = _k_post(out_heads, r16, tp,
                    lp["gat_b"].reshape(1, HID),
                    lp["ln_g"].reshape(1, HID),
                    lp["ln_b"].reshape(1, HID))
        alphas.append(al16[:E2, :HEADS])

    w1p = jnp.pad(params["c1_W"], ((0, 0), (0, HID - params["c1_W"].shape[1])))
    b1p = jnp.pad(params["c1_b"], (0, HID - params["c1_b"].shape[0]))
    w2p = jnp.pad(params["c2_W"],
                  ((0, HID - params["c2_W"].shape[0]),
                   (0, HID - params["c2_W"].shape[1])))
    logits = _k_cls(h, w1p, b1p.reshape(1, HID), w2p)[:, :2]
    return (logits, *alphas)


# speedup vs baseline: 13.8988x; 1.0098x over previous
import functools
import jax
import jax.numpy as jnp
from jax import lax
from jax.experimental import pallas as pl
from jax.experimental.pallas import tpu as pltpu
from jax.experimental.pallas import tpu_sc as plsc

N = 10000
E = 160000
E2 = E + N
D_IN = 128
HID = 128
HEADS = 8
NP = 10240
NB = 512
NBLK = NP // NB
NC = 2
NS = 16
CH = 128
E2P = 172032
PER_TILE_32 = E2P // (NC * NS)
PER_TILE_16 = E2P // NS

_HIGH = jax.lax.Precision.HIGHEST


def _lrelu(v):
    return jnp.where(v > 0, v, 0.2 * v)


def _kin_body(x_ref, w_ref, b_ref, o_ref):
    o_ref[...] = jax.nn.relu(
        jnp.dot(x_ref[...], w_ref[...], precision=_HIGH) + b_ref[...])


def _k_in(xp, w, b):
    return pl.pallas_call(
        _kin_body,
        grid=(NBLK,),
        in_specs=[
            pl.BlockSpec((NB, D_IN), lambda i: (i, 0)),
            pl.BlockSpec((D_IN, HID), lambda i: (0, 0)),
            pl.BlockSpec((1, HID), lambda i: (0, 0)),
        ],
        out_specs=pl.BlockSpec((NB, HID), lambda i: (i, 0)),
        out_shape=jax.ShapeDtypeStruct((NP, HID), jnp.float32),
    )(xp, w, b)


def _kpre_body(h_ref, gw_ref, as_ref, ad_ref, tw_ref, tb_ref,
               hh_ref, at_s_ref, at_d_ref, tp_ref, mx_ref):
    i = pl.program_id(0)
    hh = jnp.dot(h_ref[...], gw_ref[...], precision=_HIGH)
    hh_ref[...] = hh
    a_s = jnp.dot(hh, as_ref[...], precision=_HIGH)
    a_d = jnp.dot(hh, ad_ref[...], precision=_HIGH)
    at_s_ref[...] = a_s
    at_d_ref[...] = a_d
    tp_ref[...] = jnp.dot(h_ref[...], tw_ref[...], precision=_HIGH) + tb_ref[...]
    bm = jnp.max(a_s, axis=0, keepdims=True)

    @pl.when(i == 0)
    def _():
        mx_ref[...] = bm

    @pl.when(i > 0)
    def _():
        mx_ref[...] = jnp.maximum(mx_ref[...], bm)


def _k_pre(h, gat_w, as16, ad16, tp_w, tp_b):
    return pl.pallas_call(
        _kpre_body,
        grid=(NBLK,),
        in_specs=[
            pl.BlockSpec((NB, HID), lambda i: (i, 0)),
            pl.BlockSpec((HID, HEADS * HID), lambda i: (0, 0)),
            pl.BlockSpec((HEADS * HID, 16), lambda i: (0, 0)),
            pl.BlockSpec((HEADS * HID, 16), lambda i: (0, 0)),
            pl.BlockSpec((HID, HID), lambda i: (0, 0)),
            pl.BlockSpec((1, HID), lambda i: (0, 0)),
        ],
        out_specs=[
            pl.BlockSpec((NB, HEADS * HID), lambda i: (i, 0)),
            pl.BlockSpec((NB, 16), lambda i: (i, 0)),
            pl.BlockSpec((NB, 16), lambda i: (i, 0)),
            pl.BlockSpec((NB, HID), lambda i: (i, 0)),
            pl.BlockSpec((1, 16), lambda i: (0, 0)),
        ],
        out_shape=[
            jax.ShapeDtypeStruct((NP, HEADS * HID), jnp.float32),
            jax.ShapeDtypeStruct((NP, 16), jnp.float32),
            jax.ShapeDtypeStruct((NP, 16), jnp.float32),
            jax.ShapeDtypeStruct((NP, HID), jnp.float32),
            jax.ShapeDtypeStruct((1, 16), jnp.float32),
        ],
    )(h, gat_w, as16, ad16, tp_w, tp_b)


def _kcomb_body(s_ref, r_ref):
    r_ref[...] = 1.0 / (s_ref[0] + s_ref[1] + 1e-16)


def _k_combine(s2):
    return pl.pallas_call(
        _kcomb_body,
        out_shape=jax.ShapeDtypeStruct((NP, 16), jnp.float32),
    )(s2)


def _kpost_body(oh_ref, r_ref, tp_ref, gb_ref, lg_ref, lb_ref, o_ref):
    acc = jnp.zeros((NB, HID), jnp.float32)
    for hd in range(HEADS):
        acc = acc + oh_ref[hd] * r_ref[:, hd][:, None]
    g = acc * (1.0 / HEADS) + gb_ref[...]
    z = g + tp_ref[...]
    mu = jnp.mean(z, axis=-1, keepdims=True)
    zc = z - mu
    var = jnp.mean(zc * zc, axis=-1, keepdims=True)
    o_ref[...] = jax.nn.relu(zc / jnp.sqrt(var + 1e-5) * lg_ref[...] + lb_ref[...])


def _k_post(out_heads, r, tp, gb, lg, lb):
    return pl.pallas_call(
        _kpost_body,
        grid=(NBLK,),
        in_specs=[
            pl.BlockSpec((HEADS, NB, HID), lambda i: (0, i, 0)),
            pl.BlockSpec((NB, 16), lambda i: (i, 0)),
            pl.BlockSpec((NB, HID), lambda i: (i, 0)),
            pl.BlockSpec((1, HID), lambda i: (0, 0)),
            pl.BlockSpec((1, HID), lambda i: (0, 0)),
            pl.BlockSpec((1, HID), lambda i: (0, 0)),
        ],
        out_specs=pl.BlockSpec((NB, HID), lambda i: (i, 0)),
        out_shape=jax.ShapeDtypeStruct((NP, HID), jnp.float32),
    )(out_heads, r, tp, gb, lg, lb)


def _kcls_body(h_ref, w1_ref, b1_ref, w2_ref, o_ref, acc_ref):
    i = pl.program_id(0)

    @pl.when(i == 0)
    def _():
        acc_ref[...] = jnp.zeros_like(acc_ref)

    rows = i * NB + lax.broadcasted_iota(jnp.int32, (NB, 1), 0)
    hm = jnp.where(rows < N, h_ref[...], 0.0)
    acc_ref[...] = acc_ref[...] + jnp.sum(hm, axis=0, keepdims=True)

    hg = acc_ref[...] * (1.0 / N)
    z = jax.nn.relu(jnp.dot(hg, w1_ref[...], precision=_HIGH) + b1_ref[...])
    o_ref[...] = jnp.dot(z, w2_ref[...], precision=_HIGH)


def _k_cls(h, w1p, b1p, w2p):
    return pl.pallas_call(
        _kcls_body,
        grid=(NBLK,),
        in_specs=[
            pl.BlockSpec((NB, HID), lambda i: (i, 0)),
            pl.BlockSpec((HID, HID), lambda i: (0, 0)),
            pl.BlockSpec((1, HID), lambda i: (0, 0)),
            pl.BlockSpec((HID, HID), lambda i: (0, 0)),
        ],
        out_specs=pl.BlockSpec((1, HID), lambda i: (0, 0)),
        out_shape=jax.ShapeDtypeStruct((1, HID), jnp.float32),
        scratch_shapes=[pltpu.VMEM((1, HID), jnp.float32)],
    )(h, w1p, b1p, w2p)


_MESH = plsc.VectorSubcoreMesh(
    core_axis_name="c", subcore_axis_name="s", num_cores=NC, num_subcores=NS)
_SC_PARAMS = pltpu.CompilerParams(
    use_tc_tiling_on_sc=False, needs_layout_passes=False)


def _sc_stats_body(as_hbm, ad_hbm, mx_hbm, src_hbm, dst_hbm,
                   ex_hbm, s2_hbm,
                   src_v, dst_v, asr_v, adr_v, exr_v, zb_v, mx_v,
                   s_acc, sem_a, sem_b):
    cid = lax.axis_index("c")
    sid = lax.axis_index("s")
    wid = sid * NC + cid
    tile_rows = NP // NS

    for j in range(CH):
        zb_v[j, :] = jnp.zeros((16,), jnp.float32)
    for j in range(tile_rows // CH):
        pltpu.sync_copy(zb_v, s_acc.at[pl.ds(sid * tile_rows + j * CH, CH)])
    pltpu.sync_copy(mx_hbm, mx_v)
    plsc.subcore_barrier()

    base = wid * PER_TILE_32
    nchunk = PER_TILE_32 // CH

    def chunk(ci, _):
        off = base + ci * CH
        pltpu.sync_copy(src_hbm.at[pl.ds(off, CH)], src_v)
        pltpu.sync_copy(dst_hbm.at[pl.ds(off, CH)], dst_v)
        ca = pltpu.async_copy(as_hbm.at[src_v], asr_v, sem_a)
        cb = pltpu.async_copy(ad_hbm.at[dst_v], adr_v, sem_b)
        ca.wait()
        cb.wait()

        def edge(e, _):
            a = asr_v[e, :]
            b = adr_v[e, :]
            ex = jnp.exp(_lrelu(a + b) - _lrelu(mx_v[:] + b))
            exr_v[e, :] = ex
            return 0

        lax.fori_loop(0, CH, edge, 0, unroll=8)
        pltpu.sync_copy(exr_v, ex_hbm.at[pl.ds(off, CH), :])
        pltpu.sync_copy(exr_v, s_acc.at[dst_v], add=True)
        return 0

    lax.fori_loop(0, nchunk, chunk, 0)
    plsc.subcore_barrier()

    pltpu.sync_copy(
        s_acc.at[pl.ds(sid * tile_rows, tile_rows)],
        s2_hbm.at[cid, pl.ds(sid * tile_rows, tile_rows), :])


def _sc_stats(as16, ad16, mx16, srcp, dstp):
    f = pl.kernel(
        _sc_stats_body,
        out_type=[
            jax.ShapeDtypeStruct((E2P, 16), jnp.float32),
            jax.ShapeDtypeStruct((NC, NP, 16), jnp.float32),
        ],
        mesh=_MESH,
        compiler_params=_SC_PARAMS,
        scratch_types=[
            pltpu.VMEM((CH,), jnp.int32),
            pltpu.VMEM((CH,), jnp.int32),
            pltpu.VMEM((CH, 16), jnp.float32),
            pltpu.VMEM((CH, 16), jnp.float32),
            pltpu.VMEM((CH, 16), jnp.float32),
            pltpu.VMEM((CH, 16), jnp.float32),
            pltpu.VMEM((16,), jnp.float32),
            pltpu.VMEM_SHARED((NP, 16), jnp.float32),
            pltpu.SemaphoreType.DMA,
            pltpu.SemaphoreType.DMA,
        ],
    )
    return f(as16, ad16, mx16, srcp, dstp)


_NCH = PER_TILE_16 // CH
_NCHH = _NCH // 2


def _sc_agg_body(gidx_hbm, dst2_hbm, hh8_hbm, ex_hbm, r_hbm,
                 al_hbm, oh_hbm,
                 dst_v, gidxall_v, dstall_v, rows0_v, rows1_v,
                 ex0_v, ex1_v,
                 acc, sem_g0, sem_g1, sem_x0, sem_x1, sem_s0, sem_s1, sem_a):
    cid = lax.axis_index("c")
    sid = lax.axis_index("s")
    wid = sid * NC + cid
    tile_rows = NP // NS
    rows = (rows0_v, rows1_v)
    exs = (ex0_v, ex1_v)
    sgs = (sem_g0, sem_g1)
    sxs = (sem_x0, sem_x1)
    sss = (sem_s0, sem_s1)

    base = wid * PER_TILE_32

    def achunk(ci, _):
        off = base + ci * CH
        pltpu.sync_copy(dst2_hbm.at[wid * (PER_TILE_32 // CH) + ci], dst_v)
        rg = pltpu.async_copy(r_hbm.at[dst_v], ex1_v, sem_a)
        pltpu.sync_copy(ex_hbm.at[pl.ds(off, CH), :], ex0_v)
        rg.wait()

        def edge(e, _):
            ex0_v[e, :] = ex0_v[e, :] * ex1_v[e, :]
            return 0

        lax.fori_loop(0, CH, edge, 0, unroll=8)
        pltpu.sync_copy(ex0_v, al_hbm.at[pl.ds(off, CH), :])
        return 0

    lax.fori_loop(0, PER_TILE_32 // CH, achunk, 0)

    for hl in range(HEADS // NC):
        h_abs = cid * (HEADS // NC) + hl

        def zfill(j, _):
            for q in range(8):
                rows0_v[j, pl.ds(q * 16, 16)] = jnp.zeros((16,), jnp.float32)
            return 0

        lax.fori_loop(0, CH, zfill, 0, unroll=8)

        def zrow(j, _):
            pltpu.async_copy(
                rows0_v, acc.at[pl.ds(sid * tile_rows + j * CH, CH)], sem_a)
            return 0

        lax.fori_loop(0, tile_rows // CH, zrow, 0)

        def zdrain(j, _):
            pltpu.make_async_copy(
                rows0_v, acc.at[pl.ds(sid * tile_rows, CH)], sem_a).wait()
            return 0

        lax.fori_loop(0, tile_rows // CH, zdrain, 0)
        plsc.subcore_barrier()

        for half in range(2):
            pltpu.sync_copy(
                gidx_hbm.at[h_abs, pl.ds(sid * _NCH + half * _NCHH, _NCHH), :],
                gidxall_v)
            pltpu.sync_copy(
                dst2_hbm.at[pl.ds(sid * _NCH + half * _NCHH, _NCHH), :],
                dstall_v)

            ebase = sid * PER_TILE_16 + half * _NCHH * CH

            pltpu.async_copy(hh8_hbm.at[gidxall_v.at[0]], rows0_v, sem_g0)
            pltpu.async_copy(ex_hbm.at[pl.ds(ebase, CH), :], ex0_v, sem_x0)

            @pl.loop(0, _NCHH, step=2)
            def _chunks(g):
                for b in range(2):
                    ci = g + b
                    nb = 1 - b

                    @pl.when(ci + 1 < _NCHH)
                    def _():
                        @pl.when(ci >= 1)
                        def _():
                            pltpu.make_async_copy(
                                rows[nb], acc.at[dstall_v.at[0]],
                                sss[nb]).wait()
                        pltpu.async_copy(
                            hh8_hbm.at[gidxall_v.at[ci + 1]], rows[nb],
                            sgs[nb])
                        pltpu.async_copy(
                            ex_hbm.at[pl.ds(ebase + (ci + 1) * CH, CH), :],
                            exs[nb], sxs[nb])

                    pltpu.make_async_copy(
                        hh8_hbm.at[gidxall_v.at[ci]], rows[b], sgs[b]).wait()
                    pltpu.make_async_copy(
                        ex_hbm.at[pl.ds(ebase + ci * CH, CH), :],
                        exs[b], sxs[b]).wait()

                    rv = rows[b]
                    ev = exs[b]

                    def edge(e, _):
                        w = plsc.load_gather(
                            ev,
                            [jnp.full((16,), e, jnp.int32),
                             jnp.full((16,), h_abs, jnp.int32)])
                        for q in range(HID // 16):
                            rv[e, pl.ds(q * 16, 16)] = (
                                rv[e, pl.ds(q * 16, 16)] * w)
                        return 0

                    lax.fori_loop(0, CH, edge, 0, unroll=8)
                    pltpu.async_copy(
                        rv, acc.at[dstall_v.at[ci]], sss[b], add=True)

            pltpu.make_async_copy(
                rows0_v, acc.at[dstall_v.at[0]], sem_s0).wait()
            pltpu.make_async_copy(
                rows1_v, acc.at[dstall_v.at[0]], sem_s1).wait()

        plsc.subcore_barrier()

        pltpu.sync_copy(
            acc.at[pl.ds(sid * tile_rows, tile_rows)],
            oh_hbm.at[h_abs, pl.ds(sid * tile_rows, tile_rows), :])
        plsc.subcore_barrier()


def _sc_agg(gidx_all, dst2d, hh8, ex16, r16):
    f = pl.kernel(
        _sc_agg_body,
        out_type=[
            jax.ShapeDtypeStruct((E2P, 16), jnp.float32),
            jax.ShapeDtypeStruct((HEADS, NP, HID), jnp.float32),
        ],
        mesh=_MESH,
        compiler_params=_SC_PARAMS,
        scratch_types=[
            pltpu.VMEM((CH,), jnp.int32),
            pltpu.VMEM((_NCHH, CH), jnp.int32),
            pltpu.VMEM((_NCHH, CH), jnp.int32),
            pltpu.VMEM((CH, HID), jnp.float32),
            pltpu.VMEM((CH, HID), jnp.float32),
            pltpu.VMEM((CH, 16), jnp.float32),
            pltpu.VMEM((CH, 16), jnp.float32),
            pltpu.VMEM_SHARED((NP, HID), jnp.float32),
            pltpu.SemaphoreType.DMA,
            pltpu.SemaphoreType.DMA,
            pltpu.SemaphoreType.DMA,
            pltpu.SemaphoreType.DMA,
            pltpu.SemaphoreType.DMA,
            pltpu.SemaphoreType.DMA,
            pltpu.SemaphoreType.DMA,
        ],
    )
    return f(gidx_all, dst2d, hh8, ex16, r16)


def kernel(x, edge_index, params):
    loop = jnp.arange(N, dtype=edge_index.dtype)
    src = jnp.concatenate([edge_index[0], loop])
    dst = jnp.concatenate([edge_index[1], loop])
    pad = jnp.full((E2P - E2,), N, jnp.int32)
    srcp = jnp.concatenate([src.astype(jnp.int32), pad])
    dstp = jnp.concatenate([dst.astype(jnp.int32), pad])

    xp = jnp.pad(x, ((0, NP - N), (0, 0)))
    eye16 = jnp.eye(HEADS, 16, dtype=jnp.float32)
    gidx_all = (srcp[None, :] * HEADS
                + jnp.arange(HEADS, dtype=jnp.int32)[:, None]).reshape(
                    HEADS, E2P // CH, CH)
    dst2d = dstp.reshape(E2P // CH, CH)

    h = _k_in(xp, params["in_W"], params["in_b"].reshape(1, HID))

    alphas = []
    for lp in params["layers"]:
        as16 = jnp.einsum("hd,hk->hdk", lp["att_src"], eye16).reshape(
            HEADS * HID, 16)
        ad16 = jnp.einsum("hd,hk->hdk", lp["att_dst"], eye16).reshape(
            HEADS * HID, 16)
        hh, at_s, at_d, tp, mx = _k_pre(
            h, lp["gat_W"], as16, ad16, lp["tp_W"], lp["tp_b"].reshape(1, HID))
        ex16, s2 = _sc_stats(at_s, at_d, mx.reshape(16), srcp, dstp)
        r16 = _k_combine(s2)
        al16, out_heads = _sc_agg(
            gidx_all, dst2d, hh.reshape(NP * HEADS, HID), ex16, r16)
        h = _k_post(out_heads, r16, tp,
                    lp["gat_b"].reshape(1, HID),
                    lp["ln_g"].reshape(1, HID),
                    lp["ln_b"].reshape(1, HID))
        alphas.append(al16[:E2, :HEADS])

    w1p = jnp.pad(params["c1_W"], ((0, 0), (0, HID - params["c1_W"].shape[1])))
    b1p = jnp.pad(params["c1_b"], (0, HID - params["c1_b"].shape[0]))
    w2p = jnp.pad(params["c2_W"],
                  ((0, HID - params["c2_W"].shape[0]),
                   (0, HID - params["c2_W"].shape[1])))
    logits = _k_cls(h, w1p, b1p.reshape(1, HID), w2p)[:, :2]
    return (logits, *alphas)
